# baseline (speedup 1.0000x reference)
import functools

import jax
import jax.numpy as jnp
from jax import lax
from jax.experimental import pallas as pl
from jax.experimental.pallas import tpu as pltpu
from jax.experimental.pallas import tpu_sc as plsc

NC = 2
NS = 16
L = 16

C = 2048
SPLITS = 8
NB = 4
UNROLL = 16
RCH = 128


def _make_sc_kernel(B, N, M, E):
    E8 = -(-E // (SPLITS * C)) * C
    last_len = E - (SPLITS - 1) * E8
    last_r = last_len - (last_len // C) * C
    assert last_r == 0 or last_r <= C - 8
    MR = M // L
    nrt = NB * MR // RCH
    mesh = plsc.VectorSubcoreMesh(core_axis_name="c", subcore_axis_name="s")

    @functools.partial(
        pl.kernel,
        out_type=jax.ShapeDtypeStruct((B, MR, L), jnp.float32),
        mesh=mesh,
        compiler_params=pltpu.CompilerParams(
            needs_layout_passes=False, use_tc_tiling_on_sc=False),
        scratch_types=[
            pltpu.VMEM((N,), jnp.int32),
            pltpu.VMEM((N,), jnp.int32),
            pltpu.VMEM((MR, L), jnp.float32),
            pltpu.VMEM((MR, L), jnp.float32),
            pltpu.VMEM((MR, L), jnp.float32),
            pltpu.VMEM((MR, L), jnp.float32),
            pltpu.VMEM((2, C), jnp.int32),
            pltpu.VMEM((2, C), jnp.int32),
            pltpu.VMEM((2, C), jnp.float32),
            pltpu.VMEM((nrt, RCH), jnp.int32),
            pltpu.SemaphoreType.DMA((2,)),
            pltpu.SemaphoreType.DMA,
            pltpu.VMEM_SHARED((NB * MR, L), jnp.float32),
            pltpu.VMEM_SHARED((NB * MR, L), jnp.float32),
        ],
    )
    def body(xph, srch, dsth, wh, biash, idnh, out, xp0, xp1, a0, a1, a2, a3,
             sv, dv, wv, idv, sems, rsem, shr0, shr1):
        c = lax.axis_index("c")
        s = lax.axis_index("s")
        g = s // SPLITS
        h = s % SPLITS
        b0 = c * (2 * NB) + g * NB
        k0 = b0 // 2
        is_owner = h == 0
        start = h * E8
        end = jnp.minimum(start + E8, E)
        nf = (end - start) // C
        r = (end - start) - nf * C
        accs = (a0, a1, a2, a3)

        def start_chunk(slot, off):
            off = pl.multiple_of(off, 8)
            pltpu.async_copy(srch.at[pl.ds(off, C)], sv.at[slot],
                             sems.at[slot])
            pltpu.async_copy(dsth.at[pl.ds(off, C)], dv.at[slot],
                             sems.at[slot])
            pltpu.async_copy(wh.at[pl.ds(off, C)], wv.at[slot], sems.at[slot])

        def wait_chunk(slot):
            pltpu.make_async_copy(srch.at[pl.ds(0, C)], sv.at[slot],
                                  sems.at[slot]).wait()
            pltpu.make_async_copy(dsth.at[pl.ds(0, C)], dv.at[slot],
                                  sems.at[slot]).wait()
            pltpu.make_async_copy(wh.at[pl.ds(0, C)], wv.at[slot],
                                  sems.at[slot]).wait()

        def compute(sl, j, mask):
            o = pl.ds(j * L, L)
            isrc = sv[sl, o]
            idst = dv[sl, o]
            w = wv[sl, o]
            irow = idst >> 4
            icol = idst & 0xF
            xw0 = plsc.load_gather(xp0, [isrc], mask=mask)
            xw1 = plsc.load_gather(xp1, [isrc], mask=mask)
            f0 = plsc.bitcast(xw0 << 16, jnp.float32)
            f1 = plsc.bitcast(xw0 & -65536, jnp.float32)
            f2 = plsc.bitcast(xw1 << 16, jnp.float32)
            f3 = plsc.bitcast(xw1 & -65536, jnp.float32)
            plsc.addupdate_scatter(a0, [irow, icol], w * f0, mask=mask)
            plsc.addupdate_scatter(a1, [irow, icol], w * f1, mask=mask)
            plsc.addupdate_scatter(a2, [irow, icol], w * f2, mask=mask)
            plsc.addupdate_scatter(a3, [irow, icol], w * f3, mask=mask)

        start_chunk(0, start)
        pltpu.sync_copy(xph.at[k0], xp0)
        pltpu.sync_copy(xph.at[k0 + 1], xp1)
        pltpu.sync_copy(idnh, idv)

        @pl.when(jnp.logical_and(is_owner, g == 0))
        def _():
            for bb in range(NB):
                pltpu.sync_copy(biash, shr0.at[pl.ds(bb * MR, MR)])

        @pl.when(jnp.logical_and(is_owner, g == 1))
        def _():
            for bb in range(NB):
                pltpu.sync_copy(biash, shr1.at[pl.ds(bb * MR, MR)])

        zero = jnp.zeros((L,), jnp.float32)

        @plsc.parallel_loop(0, MR, unroll=4)
        def _(i):
            a0[i, :] = zero
            a1[i, :] = zero
            a2[i, :] = zero
            a3[i, :] = zero

        plsc.subcore_barrier()

        def chunk_body(gp, carry):
            for sl in range(2):
                gg = 2 * gp + sl

                @pl.when(gg + 1 < nf)
                def _():
                    start_chunk(1 - sl, start + (gg + 1) * C)

                wait_chunk(sl)

                @plsc.parallel_loop(0, C // L, unroll=UNROLL)
                def _(j):
                    compute(sl, j, None)

            return carry

        lax.fori_loop(0, nf // 2, chunk_body, 0)

        @pl.when(nf % 2 == 1)
        def _():
            wait_chunk(0)

            @plsc.parallel_loop(0, C // L, unroll=UNROLL)
            def _(j):
                compute(0, j, None)

        @pl.when(r > 0)
        def _():
            o8 = pl.multiple_of((end - C + 7) & ~7, 8)
            pltpu.sync_copy(srch.at[pl.ds(o8, C)], sv.at[1])
            pltpu.sync_copy(dsth.at[pl.ds(o8, C)], dv.at[1])
            pltpu.sync_copy(wh.at[pl.ds(o8, C)], wv.at[1])
            done = start + nf * C
            lane = lax.iota(jnp.int32, L)

            @plsc.parallel_loop(0, C // L, unroll=UNROLL)
            def _(j):
                e0 = o8 + j * L
                mask = jnp.logical_and(e0 + lane >= done, e0 + lane < end)
                compute(1, j, mask)

        def reduce_into(shr):
            copies = []
            for t in range(nrt):
                bb = t // (MR // RCH)
                r0 = (t % (MR // RCH)) * RCH
                copies.append(pltpu.async_copy(
                    accs[bb].at[pl.ds(r0, RCH)], shr.at[idv.at[t]], rsem,
                    add=True))
            for cp in copies:
                cp.wait()

        @pl.when(g == 0)
        def _():
            reduce_into(shr0)

        @pl.when(g == 1)
        def _():
            reduce_into(shr1)

        plsc.subcore_barrier()

        @pl.when(jnp.logical_and(is_owner, g == 0))
        def _():
            for bb in range(NB):
                pltpu.sync_copy(shr0.at[pl.ds(bb * MR, MR)], out.at[b0 + bb])

        @pl.when(jnp.logical_and(is_owner, g == 1))
        def _():
            for bb in range(NB):
                pltpu.sync_copy(shr1.at[pl.ds(bb * MR, MR)], out.at[b0 + bb])

    return body


def kernel(x, indices, values, bias):
    B, N, _ = x.shape
    M = bias.shape[0]
    E = values.shape[0]

    xb = lax.bitcast_convert_type(
        x[:, :, 0].astype(jnp.bfloat16).reshape(B // 2, 2, N), jnp.uint16
    ).astype(jnp.uint32)
    xp = lax.bitcast_convert_type(xb[:, 0] | (xb[:, 1] << 16), jnp.int32)

    MR = M // L
    idn = jnp.arange(4 * MR, dtype=jnp.int32).reshape(-1, RCH)

    out = _make_sc_kernel(B, N, M, E)(
        xp, indices[0], indices[1], values, bias.reshape(MR, L), idn)
    return out.reshape(B, M)[:, :, None]


# speedup vs baseline: 1.5022x; 1.5022x over previous
import functools

import jax
import jax.numpy as jnp
from jax import lax
from jax.experimental import pallas as pl
from jax.experimental.pallas import tpu as pltpu
from jax.experimental.pallas import tpu_sc as plsc

NC = 2
NS = 16
L = 16

C = 2048
SPLITS = 8
NB = 4
UNROLL = 8
RCH = 128


def _make_sc_kernel(B, N, M, E):
    E8 = -(-E // (SPLITS * C)) * C
    last_len = E - (SPLITS - 1) * E8
    last_r = last_len - (last_len // C) * C
    assert last_r == 0 or last_r <= C - 8
    MR = M // L
    nrt = NB * MR // RCH
    mesh = plsc.VectorSubcoreMesh(core_axis_name="c", subcore_axis_name="s")

    @functools.partial(
        pl.kernel,
        out_type=jax.ShapeDtypeStruct((B, MR, L), jnp.float32),
        mesh=mesh,
        compiler_params=pltpu.CompilerParams(
            needs_layout_passes=False, use_tc_tiling_on_sc=False),
        scratch_types=[
            pltpu.VMEM((N,), jnp.int32),
            pltpu.VMEM((N,), jnp.int32),
            pltpu.VMEM((MR, L), jnp.float32),
            pltpu.VMEM((MR, L), jnp.float32),
            pltpu.VMEM((MR, L), jnp.float32),
            pltpu.VMEM((MR, L), jnp.float32),
            pltpu.VMEM((2, C), jnp.int32),
            pltpu.VMEM((2, C), jnp.int32),
            pltpu.VMEM((2, C), jnp.float32),
            pltpu.VMEM((nrt, RCH), jnp.int32),
            pltpu.SemaphoreType.DMA((2,)),
            pltpu.SemaphoreType.DMA,
            pltpu.VMEM_SHARED((NB * MR, L), jnp.float32),
            pltpu.VMEM_SHARED((NB * MR, L), jnp.float32),
        ],
    )
    def body(xph, srch, dsth, wh, biash, idnh, out, xp0, xp1, a0, a1, a2, a3,
             sv, dv, wv, idv, sems, rsem, shr0, shr1):
        c = lax.axis_index("c")
        s = lax.axis_index("s")
        g = s // SPLITS
        h = s % SPLITS
        b0 = c * (2 * NB) + g * NB
        k0 = b0 // 2
        is_owner = h == 0
        start = h * E8
        end = jnp.minimum(start + E8, E)
        nf = (end - start) // C
        r = (end - start) - nf * C
        accs = (a0, a1, a2, a3)

        def start_chunk(slot, off):
            off = pl.multiple_of(off, 8)
            pltpu.async_copy(srch.at[pl.ds(off, C)], sv.at[slot],
                             sems.at[slot])
            pltpu.async_copy(dsth.at[pl.ds(off, C)], dv.at[slot],
                             sems.at[slot])
            pltpu.async_copy(wh.at[pl.ds(off, C)], wv.at[slot], sems.at[slot])

        def wait_chunk(slot):
            pltpu.make_async_copy(srch.at[pl.ds(0, C)], sv.at[slot],
                                  sems.at[slot]).wait()
            pltpu.make_async_copy(dsth.at[pl.ds(0, C)], dv.at[slot],
                                  sems.at[slot]).wait()
            pltpu.make_async_copy(wh.at[pl.ds(0, C)], wv.at[slot],
                                  sems.at[slot]).wait()

        def compute(sl, j, mask):
            o = pl.ds(j * L, L)
            isrc = sv[sl, o]
            idst = dv[sl, o]
            w = wv[sl, o]
            irow = idst >> 4
            icol = idst & 0xF
            xw0 = plsc.load_gather(xp0, [isrc], mask=mask)
            xw1 = plsc.load_gather(xp1, [isrc], mask=mask)
            f0 = plsc.bitcast(xw0 << 16, jnp.float32)
            f1 = plsc.bitcast(xw0 & -65536, jnp.float32)
            f2 = plsc.bitcast(xw1 << 16, jnp.float32)
            f3 = plsc.bitcast(xw1 & -65536, jnp.float32)
            plsc.addupdate_scatter(a0, [irow, icol], w * f0, mask=mask)
            plsc.addupdate_scatter(a1, [irow, icol], w * f1, mask=mask)
            plsc.addupdate_scatter(a2, [irow, icol], w * f2, mask=mask)
            plsc.addupdate_scatter(a3, [irow, icol], w * f3, mask=mask)

        start_chunk(0, start)
        pltpu.sync_copy(xph.at[k0], xp0)
        pltpu.sync_copy(xph.at[k0 + 1], xp1)
        pltpu.sync_copy(idnh, idv)

        @pl.when(jnp.logical_and(is_owner, g == 0))
        def _():
            for bb in range(NB):
                pltpu.sync_copy(biash, shr0.at[pl.ds(bb * MR, MR)])

        @pl.when(jnp.logical_and(is_owner, g == 1))
        def _():
            for bb in range(NB):
                pltpu.sync_copy(biash, shr1.at[pl.ds(bb * MR, MR)])

        zero = jnp.zeros((L,), jnp.float32)

        @plsc.parallel_loop(0, MR, unroll=4)
        def _(i):
            a0[i, :] = zero
            a1[i, :] = zero
            a2[i, :] = zero
            a3[i, :] = zero

        plsc.subcore_barrier()

        def chunk_body(gp, carry):
            for sl in range(2):
                gg = 2 * gp + sl

                @pl.when(gg + 1 < nf)
                def _():
                    start_chunk(1 - sl, start + (gg + 1) * C)

                wait_chunk(sl)

                @plsc.parallel_loop(0, C // L, unroll=UNROLL)
                def _(j):
                    compute(sl, j, None)

            return carry

        lax.fori_loop(0, nf // 2, chunk_body, 0)

        @pl.when(nf % 2 == 1)
        def _():
            wait_chunk(0)

            @plsc.parallel_loop(0, C // L, unroll=UNROLL)
            def _(j):
                compute(0, j, None)

        @pl.when(r > 0)
        def _():
            o8 = pl.multiple_of((end - C + 7) & ~7, 8)
            pltpu.sync_copy(srch.at[pl.ds(o8, C)], sv.at[1])
            pltpu.sync_copy(dsth.at[pl.ds(o8, C)], dv.at[1])
            pltpu.sync_copy(wh.at[pl.ds(o8, C)], wv.at[1])
            done = start + nf * C
            lane = lax.iota(jnp.int32, L)

            @plsc.parallel_loop(0, C // L, unroll=UNROLL)
            def _(j):
                e0 = o8 + j * L
                mask = jnp.logical_and(e0 + lane >= done, e0 + lane < end)
                compute(1, j, mask)

        def reduce_into(shr):
            copies = []
            for t in range(nrt):
                bb = t // (MR // RCH)
                r0 = (t % (MR // RCH)) * RCH
                copies.append(pltpu.async_copy(
                    accs[bb].at[pl.ds(r0, RCH)], shr.at[idv.at[t]], rsem,
                    add=True))
            for cp in copies:
                cp.wait()

        @pl.when(g == 0)
        def _():
            reduce_into(shr0)

        @pl.when(g == 1)
        def _():
            reduce_into(shr1)

        plsc.subcore_barrier()

        @pl.when(jnp.logical_and(is_owner, g == 0))
        def _():
            for bb in range(NB):
                pltpu.sync_copy(shr0.at[pl.ds(bb * MR, MR)], out.at[b0 + bb])

        @pl.when(jnp.logical_and(is_owner, g == 1))
        def _():
            for bb in range(NB):
                pltpu.sync_copy(shr1.at[pl.ds(bb * MR, MR)], out.at[b0 + bb])

    return body


_SLC_BLK = 262144


def _split_rows_tc(indices, E):
    grid = -(-E // _SLC_BLK)

    def body(iref, s_ref, d_ref):
        s_ref[...] = iref[0, :]
        d_ref[...] = iref[1, :]

    return pl.pallas_call(
        body,
        grid=(grid,),
        in_specs=[pl.BlockSpec((2, _SLC_BLK), lambda i: (0, i))],
        out_specs=[pl.BlockSpec((_SLC_BLK,), lambda i: (i,)),
                   pl.BlockSpec((_SLC_BLK,), lambda i: (i,))],
        out_shape=[jax.ShapeDtypeStruct((E,), jnp.int32),
                   jax.ShapeDtypeStruct((E,), jnp.int32)],
    )(indices)


def kernel(x, indices, values, bias):
    B, N, _ = x.shape
    M = bias.shape[0]
    E = values.shape[0]

    xb = lax.bitcast_convert_type(
        x[:, :, 0].astype(jnp.bfloat16).reshape(B // 2, 2, N), jnp.uint16
    ).astype(jnp.uint32)
    xp = lax.bitcast_convert_type(xb[:, 0] | (xb[:, 1] << 16), jnp.int32)

    MR = M // L
    idn = jnp.arange(4 * MR, dtype=jnp.int32).reshape(-1, RCH)

    src, dst = _split_rows_tc(indices, E)
    out = _make_sc_kernel(B, N, M, E)(
        xp, src, dst, values, bias.reshape(MR, L), idn)
    return out.reshape(B, M)[:, :, None]


# speedup vs baseline: 1.5260x; 1.0158x over previous
import functools

import jax
import jax.numpy as jnp
from jax import lax
from jax.experimental import pallas as pl
from jax.experimental.pallas import tpu as pltpu
from jax.experimental.pallas import tpu_sc as plsc

NC = 2
NS = 16
L = 16

C = 2048
SPLITS = 8
NB = 4
UNROLL = 4
RCH = 128


def _make_sc_kernel(B, N, M, E):
    E8 = -(-E // (SPLITS * C)) * C
    last_len = E - (SPLITS - 1) * E8
    last_r = last_len - (last_len // C) * C
    assert last_r == 0 or last_r <= C - 8
    MR = M // L
    nrt = NB * MR // RCH
    mesh = plsc.VectorSubcoreMesh(core_axis_name="c", subcore_axis_name="s")

    @functools.partial(
        pl.kernel,
        out_type=jax.ShapeDtypeStruct((B, MR, L), jnp.float32),
        mesh=mesh,
        compiler_params=pltpu.CompilerParams(
            needs_layout_passes=False, use_tc_tiling_on_sc=False),
        scratch_types=[
            pltpu.VMEM((N,), jnp.int32),
            pltpu.VMEM((N,), jnp.int32),
            pltpu.VMEM((MR, L), jnp.float32),
            pltpu.VMEM((MR, L), jnp.float32),
            pltpu.VMEM((MR, L), jnp.float32),
            pltpu.VMEM((MR, L), jnp.float32),
            pltpu.VMEM((2, C), jnp.int32),
            pltpu.VMEM((2, C), jnp.int32),
            pltpu.VMEM((2, C), jnp.float32),
            pltpu.VMEM((nrt, RCH), jnp.int32),
            pltpu.SemaphoreType.DMA((2,)),
            pltpu.SemaphoreType.DMA,
            pltpu.VMEM_SHARED((NB * MR, L), jnp.float32),
            pltpu.VMEM_SHARED((NB * MR, L), jnp.float32),
        ],
    )
    def body(xph, srch, dsth, wh, biash, idnh, out, xp0, xp1, a0, a1, a2, a3,
             sv, dv, wv, idv, sems, rsem, shr0, shr1):
        c = lax.axis_index("c")
        s = lax.axis_index("s")
        g = s // SPLITS
        h = s % SPLITS
        b0 = c * (2 * NB) + g * NB
        k0 = b0 // 2
        is_owner = h == 0
        start = h * E8
        end = jnp.minimum(start + E8, E)
        nf = (end - start) // C
        r = (end - start) - nf * C
        accs = (a0, a1, a2, a3)

        def start_chunk(slot, off):
            off = pl.multiple_of(off, 8)
            pltpu.async_copy(srch.at[pl.ds(off, C)], sv.at[slot],
                             sems.at[slot])
            pltpu.async_copy(dsth.at[pl.ds(off, C)], dv.at[slot],
                             sems.at[slot])
            pltpu.async_copy(wh.at[pl.ds(off, C)], wv.at[slot], sems.at[slot])

        def wait_chunk(slot):
            pltpu.make_async_copy(srch.at[pl.ds(0, C)], sv.at[slot],
                                  sems.at[slot]).wait()
            pltpu.make_async_copy(dsth.at[pl.ds(0, C)], dv.at[slot],
                                  sems.at[slot]).wait()
            pltpu.make_async_copy(wh.at[pl.ds(0, C)], wv.at[slot],
                                  sems.at[slot]).wait()

        def compute(sl, j, mask):
            o = pl.ds(j * L, L)
            isrc = sv[sl, o]
            idst = dv[sl, o]
            w = wv[sl, o]
            irow = idst >> 4
            icol = idst & 0xF
            xw0 = plsc.load_gather(xp0, [isrc], mask=mask)
            xw1 = plsc.load_gather(xp1, [isrc], mask=mask)
            f0 = plsc.bitcast(xw0 << 16, jnp.float32)
            f1 = plsc.bitcast(xw0 & -65536, jnp.float32)
            f2 = plsc.bitcast(xw1 << 16, jnp.float32)
            f3 = plsc.bitcast(xw1 & -65536, jnp.float32)
            plsc.addupdate_scatter(a0, [irow, icol], w * f0, mask=mask)
            plsc.addupdate_scatter(a1, [irow, icol], w * f1, mask=mask)
            plsc.addupdate_scatter(a2, [irow, icol], w * f2, mask=mask)
            plsc.addupdate_scatter(a3, [irow, icol], w * f3, mask=mask)

        start_chunk(0, start)
        pltpu.sync_copy(xph.at[k0], xp0)
        pltpu.sync_copy(xph.at[k0 + 1], xp1)
        pltpu.sync_copy(idnh, idv)

        @pl.when(jnp.logical_and(is_owner, g == 0))
        def _():
            for bb in range(NB):
                pltpu.sync_copy(biash, shr0.at[pl.ds(bb * MR, MR)])

        @pl.when(jnp.logical_and(is_owner, g == 1))
        def _():
            for bb in range(NB):
                pltpu.sync_copy(biash, shr1.at[pl.ds(bb * MR, MR)])

        zero = jnp.zeros((L,), jnp.float32)

        @plsc.parallel_loop(0, MR, unroll=4)
        def _(i):
            a0[i, :] = zero
            a1[i, :] = zero
            a2[i, :] = zero
            a3[i, :] = zero

        plsc.subcore_barrier()

        def chunk_body(gp, carry):
            for sl in range(2):
                gg = 2 * gp + sl

                @pl.when(gg + 1 < nf)
                def _():
                    start_chunk(1 - sl, start + (gg + 1) * C)

                wait_chunk(sl)

                @plsc.parallel_loop(0, C // L, unroll=UNROLL)
                def _(j):
                    compute(sl, j, None)

            return carry

        lax.fori_loop(0, nf // 2, chunk_body, 0)

        @pl.when(nf % 2 == 1)
        def _():
            wait_chunk(0)

            @plsc.parallel_loop(0, C // L, unroll=UNROLL)
            def _(j):
                compute(0, j, None)

        @pl.when(r > 0)
        def _():
            o8 = pl.multiple_of((end - C + 7) & ~7, 8)
            pltpu.sync_copy(srch.at[pl.ds(o8, C)], sv.at[1])
            pltpu.sync_copy(dsth.at[pl.ds(o8, C)], dv.at[1])
            pltpu.sync_copy(wh.at[pl.ds(o8, C)], wv.at[1])
            done = start + nf * C
            lane = lax.iota(jnp.int32, L)

            @plsc.parallel_loop(0, C // L, unroll=UNROLL)
            def _(j):
                e0 = o8 + j * L
                mask = jnp.logical_and(e0 + lane >= done, e0 + lane < end)
                compute(1, j, mask)

        def reduce_into(shr):
            copies = []
            for t in range(nrt):
                bb = t // (MR // RCH)
                r0 = (t % (MR // RCH)) * RCH
                copies.append(pltpu.async_copy(
                    accs[bb].at[pl.ds(r0, RCH)], shr.at[idv.at[t]], rsem,
                    add=True))
            for cp in copies:
                cp.wait()

        @pl.when(g == 0)
        def _():
            reduce_into(shr0)

        @pl.when(g == 1)
        def _():
            reduce_into(shr1)

        plsc.subcore_barrier()

        @pl.when(jnp.logical_and(is_owner, g == 0))
        def _():
            for bb in range(NB):
                pltpu.sync_copy(shr0.at[pl.ds(bb * MR, MR)], out.at[b0 + bb])

        @pl.when(jnp.logical_and(is_owner, g == 1))
        def _():
            for bb in range(NB):
                pltpu.sync_copy(shr1.at[pl.ds(bb * MR, MR)], out.at[b0 + bb])

    return body


_SLC_BLK = 262144


def _split_rows_tc(indices, E):
    grid = -(-E // _SLC_BLK)

    def body(iref, s_ref, d_ref):
        s_ref[...] = iref[0, :]
        d_ref[...] = iref[1, :]

    return pl.pallas_call(
        body,
        grid=(grid,),
        in_specs=[pl.BlockSpec((2, _SLC_BLK), lambda i: (0, i))],
        out_specs=[pl.BlockSpec((_SLC_BLK,), lambda i: (i,)),
                   pl.BlockSpec((_SLC_BLK,), lambda i: (i,))],
        out_shape=[jax.ShapeDtypeStruct((E,), jnp.int32),
                   jax.ShapeDtypeStruct((E,), jnp.int32)],
    )(indices)


def kernel(x, indices, values, bias):
    B, N, _ = x.shape
    M = bias.shape[0]
    E = values.shape[0]

    xb = lax.bitcast_convert_type(
        x[:, :, 0].astype(jnp.bfloat16).reshape(B // 2, 2, N), jnp.uint16
    ).astype(jnp.uint32)
    xp = lax.bitcast_convert_type(xb[:, 0] | (xb[:, 1] << 16), jnp.int32)

    MR = M // L
    idn = jnp.arange(4 * MR, dtype=jnp.int32).reshape(-1, RCH)

    src, dst = _split_rows_tc(indices, E)
    out = _make_sc_kernel(B, N, M, E)(
        xp, src, dst, values, bias.reshape(MR, L), idn)
    return out.reshape(B, M)[:, :, None]


# speedup vs baseline: 1.5301x; 1.0027x over previous
import functools

import jax
import jax.numpy as jnp
from jax import lax
from jax.experimental import pallas as pl
from jax.experimental.pallas import tpu as pltpu
from jax.experimental.pallas import tpu_sc as plsc

NC = 2
NS = 16
L = 16

C = 2048
SPLITS = 8
NB = 4
UNROLL = 2
RCH = 128


def _make_sc_kernel(B, N, M, E):
    E8 = -(-E // (SPLITS * C)) * C
    last_len = E - (SPLITS - 1) * E8
    last_r = last_len - (last_len // C) * C
    assert last_r == 0 or last_r <= C - 8
    MR = M // L
    nrt = NB * MR // RCH
    mesh = plsc.VectorSubcoreMesh(core_axis_name="c", subcore_axis_name="s")

    @functools.partial(
        pl.kernel,
        out_type=jax.ShapeDtypeStruct((B, MR, L), jnp.float32),
        mesh=mesh,
        compiler_params=pltpu.CompilerParams(
            needs_layout_passes=False, use_tc_tiling_on_sc=False),
        scratch_types=[
            pltpu.VMEM((N,), jnp.int32),
            pltpu.VMEM((N,), jnp.int32),
            pltpu.VMEM((MR, L), jnp.float32),
            pltpu.VMEM((MR, L), jnp.float32),
            pltpu.VMEM((MR, L), jnp.float32),
            pltpu.VMEM((MR, L), jnp.float32),
            pltpu.VMEM((2, C), jnp.int32),
            pltpu.VMEM((2, C), jnp.int32),
            pltpu.VMEM((2, C), jnp.float32),
            pltpu.VMEM((nrt, RCH), jnp.int32),
            pltpu.SemaphoreType.DMA((2,)),
            pltpu.SemaphoreType.DMA,
            pltpu.VMEM_SHARED((NB * MR, L), jnp.float32),
            pltpu.VMEM_SHARED((NB * MR, L), jnp.float32),
        ],
    )
    def body(xph, srch, dsth, wh, biash, idnh, out, xp0, xp1, a0, a1, a2, a3,
             sv, dv, wv, idv, sems, rsem, shr0, shr1):
        c = lax.axis_index("c")
        s = lax.axis_index("s")
        g = s // SPLITS
        h = s % SPLITS
        b0 = c * (2 * NB) + g * NB
        k0 = b0 // 2
        is_owner = h == 0
        start = h * E8
        end = jnp.minimum(start + E8, E)
        nf = (end - start) // C
        r = (end - start) - nf * C
        accs = (a0, a1, a2, a3)

        def start_chunk(slot, off):
            off = pl.multiple_of(off, 8)
            pltpu.async_copy(srch.at[pl.ds(off, C)], sv.at[slot],
                             sems.at[slot])
            pltpu.async_copy(dsth.at[pl.ds(off, C)], dv.at[slot],
                             sems.at[slot])
            pltpu.async_copy(wh.at[pl.ds(off, C)], wv.at[slot], sems.at[slot])

        def wait_chunk(slot):
            pltpu.make_async_copy(srch.at[pl.ds(0, C)], sv.at[slot],
                                  sems.at[slot]).wait()
            pltpu.make_async_copy(dsth.at[pl.ds(0, C)], dv.at[slot],
                                  sems.at[slot]).wait()
            pltpu.make_async_copy(wh.at[pl.ds(0, C)], wv.at[slot],
                                  sems.at[slot]).wait()

        def compute(sl, j, mask):
            o = pl.ds(j * L, L)
            isrc = sv[sl, o]
            idst = dv[sl, o]
            w = wv[sl, o]
            irow = idst >> 4
            icol = idst & 0xF
            xw0 = plsc.load_gather(xp0, [isrc], mask=mask)
            xw1 = plsc.load_gather(xp1, [isrc], mask=mask)
            f0 = plsc.bitcast(xw0 << 16, jnp.float32)
            f1 = plsc.bitcast(xw0 & -65536, jnp.float32)
            f2 = plsc.bitcast(xw1 << 16, jnp.float32)
            f3 = plsc.bitcast(xw1 & -65536, jnp.float32)
            plsc.addupdate_scatter(a0, [irow, icol], w * f0, mask=mask)
            plsc.addupdate_scatter(a1, [irow, icol], w * f1, mask=mask)
            plsc.addupdate_scatter(a2, [irow, icol], w * f2, mask=mask)
            plsc.addupdate_scatter(a3, [irow, icol], w * f3, mask=mask)

        start_chunk(0, start)
        pltpu.sync_copy(xph.at[k0], xp0)
        pltpu.sync_copy(xph.at[k0 + 1], xp1)
        pltpu.sync_copy(idnh, idv)

        @pl.when(jnp.logical_and(is_owner, g == 0))
        def _():
            for bb in range(NB):
                pltpu.sync_copy(biash, shr0.at[pl.ds(bb * MR, MR)])

        @pl.when(jnp.logical_and(is_owner, g == 1))
        def _():
            for bb in range(NB):
                pltpu.sync_copy(biash, shr1.at[pl.ds(bb * MR, MR)])

        zero = jnp.zeros((L,), jnp.float32)

        @plsc.parallel_loop(0, MR, unroll=4)
        def _(i):
            a0[i, :] = zero
            a1[i, :] = zero
            a2[i, :] = zero
            a3[i, :] = zero

        plsc.subcore_barrier()

        def chunk_body(gp, carry):
            for sl in range(2):
                gg = 2 * gp + sl

                @pl.when(gg + 1 < nf)
                def _():
                    start_chunk(1 - sl, start + (gg + 1) * C)

                wait_chunk(sl)

                @plsc.parallel_loop(0, C // L, unroll=UNROLL)
                def _(j):
                    compute(sl, j, None)

            return carry

        lax.fori_loop(0, nf // 2, chunk_body, 0)

        @pl.when(nf % 2 == 1)
        def _():
            wait_chunk(0)

            @plsc.parallel_loop(0, C // L, unroll=UNROLL)
            def _(j):
                compute(0, j, None)

        @pl.when(r > 0)
        def _():
            o8 = pl.multiple_of((end - C + 7) & ~7, 8)
            pltpu.sync_copy(srch.at[pl.ds(o8, C)], sv.at[1])
            pltpu.sync_copy(dsth.at[pl.ds(o8, C)], dv.at[1])
            pltpu.sync_copy(wh.at[pl.ds(o8, C)], wv.at[1])
            done = start + nf * C
            lane = lax.iota(jnp.int32, L)

            @plsc.parallel_loop(0, C // L, unroll=UNROLL)
            def _(j):
                e0 = o8 + j * L
                mask = jnp.logical_and(e0 + lane >= done, e0 + lane < end)
                compute(1, j, mask)

        def reduce_into(shr):
            copies = []
            for t in range(nrt):
                bb = t // (MR // RCH)
                r0 = (t % (MR // RCH)) * RCH
                copies.append(pltpu.async_copy(
                    accs[bb].at[pl.ds(r0, RCH)], shr.at[idv.at[t]], rsem,
                    add=True))
            for cp in copies:
                cp.wait()

        @pl.when(g == 0)
        def _():
            reduce_into(shr0)

        @pl.when(g == 1)
        def _():
            reduce_into(shr1)

        plsc.subcore_barrier()

        @pl.when(jnp.logical_and(is_owner, g == 0))
        def _():
            for bb in range(NB):
                pltpu.sync_copy(shr0.at[pl.ds(bb * MR, MR)], out.at[b0 + bb])

        @pl.when(jnp.logical_and(is_owner, g == 1))
        def _():
            for bb in range(NB):
                pltpu.sync_copy(shr1.at[pl.ds(bb * MR, MR)], out.at[b0 + bb])

    return body


_SLC_BLK = 262144


def _split_rows_tc(indices, E):
    grid = -(-E // _SLC_BLK)

    def body(iref, s_ref, d_ref):
        s_ref[...] = iref[0, :]
        d_ref[...] = iref[1, :]

    return pl.pallas_call(
        body,
        grid=(grid,),
        in_specs=[pl.BlockSpec((2, _SLC_BLK), lambda i: (0, i))],
        out_specs=[pl.BlockSpec((_SLC_BLK,), lambda i: (i,)),
                   pl.BlockSpec((_SLC_BLK,), lambda i: (i,))],
        out_shape=[jax.ShapeDtypeStruct((E,), jnp.int32),
                   jax.ShapeDtypeStruct((E,), jnp.int32)],
    )(indices)


def kernel(x, indices, values, bias):
    B, N, _ = x.shape
    M = bias.shape[0]
    E = values.shape[0]

    xb = lax.bitcast_convert_type(
        x[:, :, 0].astype(jnp.bfloat16).reshape(B // 2, 2, N), jnp.uint16
    ).astype(jnp.uint32)
    xp = lax.bitcast_convert_type(xb[:, 0] | (xb[:, 1] << 16), jnp.int32)

    MR = M // L
    idn = jnp.arange(4 * MR, dtype=jnp.int32).reshape(-1, RCH)

    src, dst = _split_rows_tc(indices, E)
    out = _make_sc_kernel(B, N, M, E)(
        xp, src, dst, values, bias.reshape(MR, L), idn)
    return out.reshape(B, M)[:, :, None]


# speedup vs baseline: 1.6139x; 1.0547x over previous
import functools

import jax
import jax.numpy as jnp
from jax import lax
from jax.experimental import pallas as pl
from jax.experimental.pallas import tpu as pltpu
from jax.experimental.pallas import tpu_sc as plsc

NC = 2
NS = 16
L = 16

C = 2048
SPLITS = 8
NB = 4
UNROLL = 2
RCH = 128


def _make_sc_kernel(B, N, M, E):
    E8 = -(-E // (SPLITS * C)) * C
    last_len = E - (SPLITS - 1) * E8
    last_r = last_len - (last_len // C) * C
    assert last_r == 0 or last_r <= C - 8
    MR = M // L
    nrt = NB * MR // RCH
    mesh = plsc.VectorSubcoreMesh(core_axis_name="c", subcore_axis_name="s")

    @functools.partial(
        pl.kernel,
        out_type=jax.ShapeDtypeStruct((B, MR, L), jnp.float32),
        mesh=mesh,
        compiler_params=pltpu.CompilerParams(
            needs_layout_passes=False, use_tc_tiling_on_sc=False),
        scratch_types=[
            pltpu.VMEM((N,), jnp.int32),
            pltpu.VMEM((N,), jnp.int32),
            pltpu.VMEM((MR, L), jnp.float32),
            pltpu.VMEM((MR, L), jnp.float32),
            pltpu.VMEM((MR, L), jnp.float32),
            pltpu.VMEM((MR, L), jnp.float32),
            pltpu.VMEM((2, C), jnp.int32),
            pltpu.VMEM((2, C), jnp.float32),
            pltpu.VMEM((nrt, RCH), jnp.int32),
            pltpu.SemaphoreType.DMA((2,)),
            pltpu.SemaphoreType.DMA,
            pltpu.VMEM_SHARED((NB * MR, L), jnp.float32),
            pltpu.VMEM_SHARED((NB * MR, L), jnp.float32),
        ],
    )
    def body(xph, pkh, wh, biash, idnh, out, xp0, xp1, a0, a1, a2, a3,
             pv, wv, idv, sems, rsem, shr0, shr1):
        c = lax.axis_index("c")
        s = lax.axis_index("s")
        g = s // SPLITS
        h = s % SPLITS
        b0 = c * (2 * NB) + g * NB
        k0 = b0 // 2
        is_owner = h == 0
        start = h * E8
        end = jnp.minimum(start + E8, E)
        nf = (end - start) // C
        r = (end - start) - nf * C
        accs = (a0, a1, a2, a3)

        def start_chunk(slot, off):
            off = pl.multiple_of(off, 8)
            pltpu.async_copy(pkh.at[pl.ds(off, C)], pv.at[slot],
                             sems.at[slot])
            pltpu.async_copy(wh.at[pl.ds(off, C)], wv.at[slot], sems.at[slot])

        def wait_chunk(slot):
            pltpu.make_async_copy(pkh.at[pl.ds(0, C)], pv.at[slot],
                                  sems.at[slot]).wait()
            pltpu.make_async_copy(wh.at[pl.ds(0, C)], wv.at[slot],
                                  sems.at[slot]).wait()

        def compute(sl, j, mask):
            o = pl.ds(j * L, L)
            p = pv[sl, o]
            w = wv[sl, o]
            isrc = p & 0x3FFF
            irow = p >> 18
            icol = (p >> 14) & 0xF
            xw0 = plsc.load_gather(xp0, [isrc], mask=mask)
            xw1 = plsc.load_gather(xp1, [isrc], mask=mask)
            f0 = plsc.bitcast(xw0 << 16, jnp.float32)
            f1 = plsc.bitcast(xw0 & -65536, jnp.float32)
            f2 = plsc.bitcast(xw1 << 16, jnp.float32)
            f3 = plsc.bitcast(xw1 & -65536, jnp.float32)
            plsc.addupdate_scatter(a0, [irow, icol], w * f0, mask=mask)
            plsc.addupdate_scatter(a1, [irow, icol], w * f1, mask=mask)
            plsc.addupdate_scatter(a2, [irow, icol], w * f2, mask=mask)
            plsc.addupdate_scatter(a3, [irow, icol], w * f3, mask=mask)

        start_chunk(0, start)
        pltpu.sync_copy(xph.at[k0], xp0)
        pltpu.sync_copy(xph.at[k0 + 1], xp1)
        pltpu.sync_copy(idnh, idv)

        @pl.when(jnp.logical_and(is_owner, g == 0))
        def _():
            for bb in range(NB):
                pltpu.sync_copy(biash, shr0.at[pl.ds(bb * MR, MR)])

        @pl.when(jnp.logical_and(is_owner, g == 1))
        def _():
            for bb in range(NB):
                pltpu.sync_copy(biash, shr1.at[pl.ds(bb * MR, MR)])

        zero = jnp.zeros((L,), jnp.float32)

        @plsc.parallel_loop(0, MR, unroll=4)
        def _(i):
            a0[i, :] = zero
            a1[i, :] = zero
            a2[i, :] = zero
            a3[i, :] = zero

        plsc.subcore_barrier()

        def chunk_body(gp, carry):
            for sl in range(2):
                gg = 2 * gp + sl

                @pl.when(gg + 1 < nf)
                def _():
                    start_chunk(1 - sl, start + (gg + 1) * C)

                wait_chunk(sl)

                @plsc.parallel_loop(0, C // L, unroll=UNROLL)
                def _(j):
                    compute(sl, j, None)

            return carry

        lax.fori_loop(0, nf // 2, chunk_body, 0)

        @pl.when(nf % 2 == 1)
        def _():
            wait_chunk(0)

            @plsc.parallel_loop(0, C // L, unroll=UNROLL)
            def _(j):
                compute(0, j, None)

        @pl.when(r > 0)
        def _():
            o8 = pl.multiple_of((end - C + 7) & ~7, 8)
            pltpu.sync_copy(pkh.at[pl.ds(o8, C)], pv.at[1])
            pltpu.sync_copy(wh.at[pl.ds(o8, C)], wv.at[1])
            done = start + nf * C
            lane = lax.iota(jnp.int32, L)

            @plsc.parallel_loop(0, C // L, unroll=UNROLL)
            def _(j):
                e0 = o8 + j * L
                mask = jnp.logical_and(e0 + lane >= done, e0 + lane < end)
                compute(1, j, mask)

        def reduce_into(shr):
            copies = []
            for t in range(nrt):
                bb = t // (MR // RCH)
                r0 = (t % (MR // RCH)) * RCH
                copies.append(pltpu.async_copy(
                    accs[bb].at[pl.ds(r0, RCH)], shr.at[idv.at[t]], rsem,
                    add=True))
            for cp in copies:
                cp.wait()

        @pl.when(g == 0)
        def _():
            reduce_into(shr0)

        @pl.when(g == 1)
        def _():
            reduce_into(shr1)

        plsc.subcore_barrier()

        @pl.when(jnp.logical_and(is_owner, g == 0))
        def _():
            for bb in range(NB):
                pltpu.sync_copy(shr0.at[pl.ds(bb * MR, MR)], out.at[b0 + bb])

        @pl.when(jnp.logical_and(is_owner, g == 1))
        def _():
            for bb in range(NB):
                pltpu.sync_copy(shr1.at[pl.ds(bb * MR, MR)], out.at[b0 + bb])

    return body


_SLC_BLK = 262144


def _pack_rows_tc(indices, E):
    grid = -(-E // _SLC_BLK)

    def body(iref, p_ref):
        p_ref[...] = iref[0, :] | (iref[1, :] << 14)

    return pl.pallas_call(
        body,
        grid=(grid,),
        in_specs=[pl.BlockSpec((2, _SLC_BLK), lambda i: (0, i))],
        out_specs=pl.BlockSpec((_SLC_BLK,), lambda i: (i,)),
        out_shape=jax.ShapeDtypeStruct((E,), jnp.int32),
    )(indices)


def kernel(x, indices, values, bias):
    B, N, _ = x.shape
    M = bias.shape[0]
    E = values.shape[0]

    xb = lax.bitcast_convert_type(
        x[:, :, 0].astype(jnp.bfloat16).reshape(B // 2, 2, N), jnp.uint16
    ).astype(jnp.uint32)
    xp = lax.bitcast_convert_type(xb[:, 0] | (xb[:, 1] << 16), jnp.int32)

    MR = M // L
    idn = jnp.arange(4 * MR, dtype=jnp.int32).reshape(-1, RCH)

    pk = _pack_rows_tc(indices, E)
    out = _make_sc_kernel(B, N, M, E)(
        xp, pk, values, bias.reshape(MR, L), idn)
    return out.reshape(B, M)[:, :, None]


# speedup vs baseline: 1.6172x; 1.0020x over previous
import functools

import jax
import jax.numpy as jnp
from jax import lax
from jax.experimental import pallas as pl
from jax.experimental.pallas import tpu as pltpu
from jax.experimental.pallas import tpu_sc as plsc

NC = 2
NS = 16
L = 16

C = 4096
SPLITS = 8
NB = 4
UNROLL = 2
RCH = 128


def _make_sc_kernel(B, N, M, E):
    E8 = -(-E // (SPLITS * C)) * C
    last_len = E - (SPLITS - 1) * E8
    last_r = last_len - (last_len // C) * C
    assert last_r == 0 or last_r <= C - 8
    MR = M // L
    nrt = NB * MR // RCH
    mesh = plsc.VectorSubcoreMesh(core_axis_name="c", subcore_axis_name="s")

    @functools.partial(
        pl.kernel,
        out_type=jax.ShapeDtypeStruct((B, MR, L), jnp.float32),
        mesh=mesh,
        compiler_params=pltpu.CompilerParams(
            needs_layout_passes=False, use_tc_tiling_on_sc=False),
        scratch_types=[
            pltpu.VMEM((N,), jnp.int32),
            pltpu.VMEM((N,), jnp.int32),
            pltpu.VMEM((MR, L), jnp.float32),
            pltpu.VMEM((MR, L), jnp.float32),
            pltpu.VMEM((MR, L), jnp.float32),
            pltpu.VMEM((MR, L), jnp.float32),
            pltpu.VMEM((2, C), jnp.int32),
            pltpu.VMEM((2, C), jnp.float32),
            pltpu.VMEM((nrt, RCH), jnp.int32),
            pltpu.SemaphoreType.DMA((2,)),
            pltpu.SemaphoreType.DMA,
            pltpu.VMEM_SHARED((NB * MR, L), jnp.float32),
            pltpu.VMEM_SHARED((NB * MR, L), jnp.float32),
        ],
    )
    def body(xph, pkh, wh, biash, idnh, out, xp0, xp1, a0, a1, a2, a3,
             pv, wv, idv, sems, rsem, shr0, shr1):
        c = lax.axis_index("c")
        s = lax.axis_index("s")
        g = s // SPLITS
        h = s % SPLITS
        b0 = c * (2 * NB) + g * NB
        k0 = b0 // 2
        is_owner = h == 0
        start = h * E8
        end = jnp.minimum(start + E8, E)
        nf = (end - start) // C
        r = (end - start) - nf * C
        accs = (a0, a1, a2, a3)

        def start_chunk(slot, off):
            off = pl.multiple_of(off, 8)
            pltpu.async_copy(pkh.at[pl.ds(off, C)], pv.at[slot],
                             sems.at[slot])
            pltpu.async_copy(wh.at[pl.ds(off, C)], wv.at[slot], sems.at[slot])

        def wait_chunk(slot):
            pltpu.make_async_copy(pkh.at[pl.ds(0, C)], pv.at[slot],
                                  sems.at[slot]).wait()
            pltpu.make_async_copy(wh.at[pl.ds(0, C)], wv.at[slot],
                                  sems.at[slot]).wait()

        def compute(sl, j, mask):
            o = pl.ds(j * L, L)
            p = pv[sl, o]
            w = wv[sl, o]
            isrc = p & 0x3FFF
            irow = p >> 18
            icol = (p >> 14) & 0xF
            xw0 = plsc.load_gather(xp0, [isrc], mask=mask)
            xw1 = plsc.load_gather(xp1, [isrc], mask=mask)
            f0 = plsc.bitcast(xw0 << 16, jnp.float32)
            f1 = plsc.bitcast(xw0 & -65536, jnp.float32)
            f2 = plsc.bitcast(xw1 << 16, jnp.float32)
            f3 = plsc.bitcast(xw1 & -65536, jnp.float32)
            plsc.addupdate_scatter(a0, [irow, icol], w * f0, mask=mask)
            plsc.addupdate_scatter(a1, [irow, icol], w * f1, mask=mask)
            plsc.addupdate_scatter(a2, [irow, icol], w * f2, mask=mask)
            plsc.addupdate_scatter(a3, [irow, icol], w * f3, mask=mask)

        start_chunk(0, start)
        pltpu.sync_copy(xph.at[k0], xp0)
        pltpu.sync_copy(xph.at[k0 + 1], xp1)
        pltpu.sync_copy(idnh, idv)

        @pl.when(jnp.logical_and(is_owner, g == 0))
        def _():
            for bb in range(NB):
                pltpu.sync_copy(biash, shr0.at[pl.ds(bb * MR, MR)])

        @pl.when(jnp.logical_and(is_owner, g == 1))
        def _():
            for bb in range(NB):
                pltpu.sync_copy(biash, shr1.at[pl.ds(bb * MR, MR)])

        zero = jnp.zeros((L,), jnp.float32)

        @plsc.parallel_loop(0, MR, unroll=4)
        def _(i):
            a0[i, :] = zero
            a1[i, :] = zero
            a2[i, :] = zero
            a3[i, :] = zero

        plsc.subcore_barrier()

        def chunk_body(gp, carry):
            for sl in range(2):
                gg = 2 * gp + sl

                @pl.when(gg + 1 < nf)
                def _():
                    start_chunk(1 - sl, start + (gg + 1) * C)

                wait_chunk(sl)

                @plsc.parallel_loop(0, C // L, unroll=UNROLL)
                def _(j):
                    compute(sl, j, None)

            return carry

        lax.fori_loop(0, nf // 2, chunk_body, 0)

        @pl.when(nf % 2 == 1)
        def _():
            wait_chunk(0)

            @plsc.parallel_loop(0, C // L, unroll=UNROLL)
            def _(j):
                compute(0, j, None)

        @pl.when(r > 0)
        def _():
            o8 = pl.multiple_of((end - C + 7) & ~7, 8)
            pltpu.sync_copy(pkh.at[pl.ds(o8, C)], pv.at[1])
            pltpu.sync_copy(wh.at[pl.ds(o8, C)], wv.at[1])
            done = start + nf * C
            lane = lax.iota(jnp.int32, L)

            @plsc.parallel_loop(0, C // L, unroll=UNROLL)
            def _(j):
                e0 = o8 + j * L
                mask = jnp.logical_and(e0 + lane >= done, e0 + lane < end)
                compute(1, j, mask)

        def reduce_into(shr):
            copies = []
            for t in range(nrt):
                bb = t // (MR // RCH)
                r0 = (t % (MR // RCH)) * RCH
                copies.append(pltpu.async_copy(
                    accs[bb].at[pl.ds(r0, RCH)], shr.at[idv.at[t]], rsem,
                    add=True))
            for cp in copies:
                cp.wait()

        @pl.when(g == 0)
        def _():
            reduce_into(shr0)

        @pl.when(g == 1)
        def _():
            reduce_into(shr1)

        plsc.subcore_barrier()

        @pl.when(jnp.logical_and(is_owner, g == 0))
        def _():
            for bb in range(NB):
                pltpu.sync_copy(shr0.at[pl.ds(bb * MR, MR)], out.at[b0 + bb])

        @pl.when(jnp.logical_and(is_owner, g == 1))
        def _():
            for bb in range(NB):
                pltpu.sync_copy(shr1.at[pl.ds(bb * MR, MR)], out.at[b0 + bb])

    return body


_SLC_BLK = 262144


def _pack_rows_tc(indices, E):
    grid = -(-E // _SLC_BLK)

    def body(iref, p_ref):
        p_ref[...] = iref[0, :] | (iref[1, :] << 14)

    return pl.pallas_call(
        body,
        grid=(grid,),
        in_specs=[pl.BlockSpec((2, _SLC_BLK), lambda i: (0, i))],
        out_specs=pl.BlockSpec((_SLC_BLK,), lambda i: (i,)),
        out_shape=jax.ShapeDtypeStruct((E,), jnp.int32),
    )(indices)


def kernel(x, indices, values, bias):
    B, N, _ = x.shape
    M = bias.shape[0]
    E = values.shape[0]

    xb = lax.bitcast_convert_type(
        x[:, :, 0].astype(jnp.bfloat16).reshape(B // 2, 2, N), jnp.uint16
    ).astype(jnp.uint32)
    xp = lax.bitcast_convert_type(xb[:, 0] | (xb[:, 1] << 16), jnp.int32)

    MR = M // L
    idn = jnp.arange(4 * MR, dtype=jnp.int32).reshape(-1, RCH)

    pk = _pack_rows_tc(indices, E)
    out = _make_sc_kernel(B, N, M, E)(
        xp, pk, values, bias.reshape(MR, L), idn)
    return out.reshape(B, M)[:, :, None]


# speedup vs baseline: 1.6779x; 1.0376x over previous
import functools

import jax
import jax.numpy as jnp
from jax import lax
from jax.experimental import pallas as pl
from jax.experimental.pallas import tpu as pltpu
from jax.experimental.pallas import tpu_sc as plsc

NC = 2
NS = 16
L = 16

C = 4096
SPLITS = 8
NB = 4
UNROLL = 1
RCH = 128


def _make_sc_kernel(B, N, M, E):
    E8 = -(-E // (SPLITS * C)) * C
    last_len = E - (SPLITS - 1) * E8
    last_r = last_len - (last_len // C) * C
    assert last_r == 0 or last_r <= C - 8
    MR = M // L
    nrt = NB * MR // RCH
    mesh = plsc.VectorSubcoreMesh(core_axis_name="c", subcore_axis_name="s")

    @functools.partial(
        pl.kernel,
        out_type=jax.ShapeDtypeStruct((B, MR, L), jnp.float32),
        mesh=mesh,
        compiler_params=pltpu.CompilerParams(
            needs_layout_passes=False, use_tc_tiling_on_sc=False),
        scratch_types=[
            pltpu.VMEM((N,), jnp.int32),
            pltpu.VMEM((N,), jnp.int32),
            pltpu.VMEM((MR, L), jnp.float32),
            pltpu.VMEM((MR, L), jnp.float32),
            pltpu.VMEM((MR, L), jnp.float32),
            pltpu.VMEM((MR, L), jnp.float32),
            pltpu.VMEM((2, C), jnp.int32),
            pltpu.VMEM((2, C), jnp.float32),
            pltpu.VMEM((nrt, RCH), jnp.int32),
            pltpu.SemaphoreType.DMA((2,)),
            pltpu.SemaphoreType.DMA,
            pltpu.VMEM_SHARED((NB * MR, L), jnp.float32),
            pltpu.VMEM_SHARED((NB * MR, L), jnp.float32),
        ],
    )
    def body(xph, pkh, wh, biash, idnh, out, xp0, xp1, a0, a1, a2, a3,
             pv, wv, idv, sems, rsem, shr0, shr1):
        c = lax.axis_index("c")
        s = lax.axis_index("s")
        g = s // SPLITS
        h = s % SPLITS
        b0 = c * (2 * NB) + g * NB
        k0 = b0 // 2
        is_owner = h == 0
        start = h * E8
        end = jnp.minimum(start + E8, E)
        nf = (end - start) // C
        r = (end - start) - nf * C
        accs = (a0, a1, a2, a3)

        def start_chunk(slot, off):
            off = pl.multiple_of(off, 8)
            pltpu.async_copy(pkh.at[pl.ds(off, C)], pv.at[slot],
                             sems.at[slot])
            pltpu.async_copy(wh.at[pl.ds(off, C)], wv.at[slot], sems.at[slot])

        def wait_chunk(slot):
            pltpu.make_async_copy(pkh.at[pl.ds(0, C)], pv.at[slot],
                                  sems.at[slot]).wait()
            pltpu.make_async_copy(wh.at[pl.ds(0, C)], wv.at[slot],
                                  sems.at[slot]).wait()

        def compute(sl, j, mask):
            o = pl.ds(j * L, L)
            p = pv[sl, o]
            w = wv[sl, o]
            isrc = p & 0x3FFF
            irow = p >> 18
            icol = (p >> 14) & 0xF
            xw0 = plsc.load_gather(xp0, [isrc], mask=mask)
            xw1 = plsc.load_gather(xp1, [isrc], mask=mask)
            f0 = plsc.bitcast(xw0 << 16, jnp.float32)
            f1 = plsc.bitcast(xw0 & -65536, jnp.float32)
            f2 = plsc.bitcast(xw1 << 16, jnp.float32)
            f3 = plsc.bitcast(xw1 & -65536, jnp.float32)
            plsc.addupdate_scatter(a0, [irow, icol], w * f0, mask=mask)
            plsc.addupdate_scatter(a1, [irow, icol], w * f1, mask=mask)
            plsc.addupdate_scatter(a2, [irow, icol], w * f2, mask=mask)
            plsc.addupdate_scatter(a3, [irow, icol], w * f3, mask=mask)

        start_chunk(0, start)
        pltpu.sync_copy(xph.at[k0], xp0)
        pltpu.sync_copy(xph.at[k0 + 1], xp1)
        pltpu.sync_copy(idnh, idv)

        @pl.when(jnp.logical_and(is_owner, g == 0))
        def _():
            for bb in range(NB):
                pltpu.sync_copy(biash, shr0.at[pl.ds(bb * MR, MR)])

        @pl.when(jnp.logical_and(is_owner, g == 1))
        def _():
            for bb in range(NB):
                pltpu.sync_copy(biash, shr1.at[pl.ds(bb * MR, MR)])

        zero = jnp.zeros((L,), jnp.float32)

        @plsc.parallel_loop(0, MR, unroll=4)
        def _(i):
            a0[i, :] = zero
            a1[i, :] = zero
            a2[i, :] = zero
            a3[i, :] = zero

        plsc.subcore_barrier()

        def chunk_body(gp, carry):
            for sl in range(2):
                gg = 2 * gp + sl

                @pl.when(gg + 1 < nf)
                def _():
                    start_chunk(1 - sl, start + (gg + 1) * C)

                wait_chunk(sl)

                @plsc.parallel_loop(0, C // L, unroll=UNROLL)
                def _(j):
                    compute(sl, j, None)

            return carry

        lax.fori_loop(0, nf // 2, chunk_body, 0)

        @pl.when(nf % 2 == 1)
        def _():
            wait_chunk(0)

            @plsc.parallel_loop(0, C // L, unroll=UNROLL)
            def _(j):
                compute(0, j, None)

        @pl.when(r > 0)
        def _():
            o8 = pl.multiple_of((end - C + 7) & ~7, 8)
            pltpu.sync_copy(pkh.at[pl.ds(o8, C)], pv.at[1])
            pltpu.sync_copy(wh.at[pl.ds(o8, C)], wv.at[1])
            done = start + nf * C
            lane = lax.iota(jnp.int32, L)

            @plsc.parallel_loop(0, C // L, unroll=UNROLL)
            def _(j):
                e0 = o8 + j * L
                mask = jnp.logical_and(e0 + lane >= done, e0 + lane < end)
                compute(1, j, mask)

        def reduce_into(shr):
            copies = []
            for t in range(nrt):
                bb = t // (MR // RCH)
                r0 = (t % (MR // RCH)) * RCH
                copies.append(pltpu.async_copy(
                    accs[bb].at[pl.ds(r0, RCH)], shr.at[idv.at[t]], rsem,
                    add=True))
            for cp in copies:
                cp.wait()

        @pl.when(g == 0)
        def _():
            reduce_into(shr0)

        @pl.when(g == 1)
        def _():
            reduce_into(shr1)

        plsc.subcore_barrier()

        @pl.when(jnp.logical_and(is_owner, g == 0))
        def _():
            for bb in range(NB):
                pltpu.sync_copy(shr0.at[pl.ds(bb * MR, MR)], out.at[b0 + bb])

        @pl.when(jnp.logical_and(is_owner, g == 1))
        def _():
            for bb in range(NB):
                pltpu.sync_copy(shr1.at[pl.ds(bb * MR, MR)], out.at[b0 + bb])

    return body


_SLC_BLK = 262144


def _pack_rows_tc(indices, E):
    grid = -(-E // _SLC_BLK)

    def body(iref, p_ref):
        p_ref[...] = iref[0, :] | (iref[1, :] << 14)

    return pl.pallas_call(
        body,
        grid=(grid,),
        in_specs=[pl.BlockSpec((2, _SLC_BLK), lambda i: (0, i))],
        out_specs=pl.BlockSpec((_SLC_BLK,), lambda i: (i,)),
        out_shape=jax.ShapeDtypeStruct((E,), jnp.int32),
    )(indices)


def kernel(x, indices, values, bias):
    B, N, _ = x.shape
    M = bias.shape[0]
    E = values.shape[0]

    xb = lax.bitcast_convert_type(
        x[:, :, 0].astype(jnp.bfloat16).reshape(B // 2, 2, N), jnp.uint16
    ).astype(jnp.uint32)
    xp = lax.bitcast_convert_type(xb[:, 0] | (xb[:, 1] << 16), jnp.int32)

    MR = M // L
    idn = jnp.arange(4 * MR, dtype=jnp.int32).reshape(-1, RCH)

    pk = _pack_rows_tc(indices, E)
    out = _make_sc_kernel(B, N, M, E)(
        xp, pk, values, bias.reshape(MR, L), idn)
    return out.reshape(B, M)[:, :, None]
